# Initial kernel scaffold; baseline (speedup 1.0000x reference)
import jax
import jax.numpy as jnp
from jax.experimental import pallas as pl


def kernel(feat0, feat1, params):
    raise NotImplementedError("write your pallas kernel here")


# speedup vs baseline: 1.6354x; 1.6354x over previous
import jax
import jax.numpy as jnp
from jax.experimental import pallas as pl
from jax.experimental.pallas import tpu as pltpu

C = 128
G = 8
CG = 16
EPS = 1e-5


def _round8(n):
    return ((n + 7) // 8) * 8


def _make_layer_kernel(D, W, Pp, rows, use_gn, cout):
    W2 = W + 2
    base = W2 + 1

    def kern(x_ref, w_ref, b_ref, g_ref, be_ref, o_ref):
        o_ref[...] = jnp.zeros_like(o_ref)
        mask = ((jax.lax.broadcasted_iota(jnp.int32, (rows, 1), 0) % W2) < W)
        maskf = mask.astype(jnp.float32)
        bias = b_ref[0]

        def body(d, carry):
            s_c, q_c = carry
            acc = None
            for kd in range(3):
                plane = x_ref[0, 0, d + kd]
                for kh in range(3):
                    for kw in range(3):
                        off = kh * W2 + kw
                        lhs = jax.lax.slice_in_dim(plane, off, off + rows, axis=0)
                        t = 9 * kd + 3 * kh + kw
                        pp = jnp.dot(lhs, w_ref[0, t],
                                     preferred_element_type=jnp.float32)
                        acc = pp if acc is None else acc + pp
            acc = (acc + bias) * maskf
            o_ref[0, 0, d + 1, pl.ds(base, rows), :] = acc
            s_c = s_c + jnp.sum(acc, axis=0, keepdims=True)
            q_c = q_c + jnp.sum(acc * acc, axis=0, keepdims=True)
            return s_c, q_c

        s_c, q_c = jax.lax.fori_loop(
            0, D, body,
            (jnp.zeros((1, cout), jnp.float32), jnp.zeros((1, cout), jnp.float32)))

        if use_gn:
            gi = jax.lax.broadcasted_iota(jnp.int32, (C, C), 0) // CG
            gj = jax.lax.broadcasted_iota(jnp.int32, (C, C), 1) // CG
            agg = (gi == gj).astype(jnp.float32)
            cnt = float(D * W * W * CG)
            gs = jnp.dot(s_c, agg, preferred_element_type=jnp.float32) / cnt
            gq = jnp.dot(q_c, agg, preferred_element_type=jnp.float32) / cnt
            var = gq - gs * gs
            inv = jax.lax.rsqrt(var + EPS)
            scale = g_ref[0]
            beta = be_ref[0]
            a = inv * scale
            bb = beta - gs * inv * scale

            def body2(d, _):
                y = o_ref[0, 0, d + 1, pl.ds(base, rows), :]
                y = jnp.maximum(y * a + bb, 0.0) * maskf
                o_ref[0, 0, d + 1, pl.ds(base, rows), :] = y
                return 0

            jax.lax.fori_loop(0, D, body2, 0)

    return kern


def _layer_call(x, w, b, g, be, D, W, Pp, rows, use_gn, shared_input, cout):
    B = x.shape[0]
    T = w.shape[0]
    D2 = D + 2
    kern = _make_layer_kernel(D, W, Pp, rows, use_gn, cout)
    if shared_input:
        x_spec = pl.BlockSpec((1, 1, D2, Pp, C), lambda bi, ti: (bi, 0, 0, 0, 0))
    else:
        x_spec = pl.BlockSpec((1, 1, D2, Pp, C), lambda bi, ti: (bi, ti, 0, 0, 0))
    w_spec = pl.BlockSpec((1, 27, C, cout), lambda bi, ti: (ti, 0, 0, 0))
    v_spec = pl.BlockSpec((1, 1, cout), lambda bi, ti: (ti, 0, 0))
    vC_spec = pl.BlockSpec((1, 1, C), lambda bi, ti: (ti, 0, 0))
    o_spec = pl.BlockSpec((1, 1, D2, Pp, cout), lambda bi, ti: (bi, ti, 0, 0, 0))
    return pl.pallas_call(
        kern,
        grid=(B, T),
        in_specs=[x_spec, w_spec, v_spec, vC_spec, vC_spec],
        out_specs=o_spec,
        out_shape=jax.ShapeDtypeStruct((B, T, D2, Pp, cout), jnp.float32),
        compiler_params=pltpu.CompilerParams(
            dimension_semantics=("parallel", "parallel")),
    )(x, w, b, g, be)


def _prep_x(feat, D, W, Pp):
    B = feat.shape[0]
    x = jnp.transpose(feat, (0, 2, 3, 4, 1))
    x = jnp.pad(x, ((0, 0), (1, 1), (1, 1), (1, 1), (0, 0)))
    x = x.reshape(B, D + 2, (W + 2) * (W + 2), C)
    x = jnp.pad(x, ((0, 0), (0, 0), (0, Pp - (W + 2) * (W + 2)), (0, 0)))
    return x[:, None]


def _prep_w(w):
    return jnp.transpose(w, (2, 3, 4, 1, 0)).reshape(27, w.shape[1], w.shape[0])


def _run_level(feat, params, D):
    W = D
    W2 = W + 2
    plane = W2 * W2
    rows = _round8((D - 1) * W2 + W)
    Pp = _round8(max(plane, 2 * W2 + 2 + rows))
    B = feat.shape[0]

    x = _prep_x(feat, D, W, Pp)
    pc, pr = params['cls'], params['reg']
    for l in range(4):
        wc, bc, gc, bec = pc['conv'][l]
        wr, br, gr, ber = pr['conv'][l]
        w = jnp.stack([_prep_w(wc), _prep_w(wr)])
        b = jnp.stack([bc, br])[:, None, :]
        g = jnp.stack([gc, gr])[:, None, :]
        be = jnp.stack([bec, ber])[:, None, :]
        x = _layer_call(x, w, b, g, be, D, W, Pp, rows, True, l == 0, C)

    (wco, boc), (wro, bor) = pc['out'], pr['out']
    nco, nro = wco.shape[0], wro.shape[0]
    co = 32
    wo = jnp.stack([
        jnp.pad(_prep_w(wco), ((0, 0), (0, 0), (0, co - nco))),
        jnp.pad(_prep_w(wro), ((0, 0), (0, 0), (0, co - nro))),
    ])
    bo = jnp.stack([jnp.pad(boc, (0, co - nco)), jnp.pad(bor, (0, co - nro))])
    bo = bo[:, None, :]
    dummy = jnp.zeros((2, 1, C), jnp.float32)
    o = _layer_call(x, wo, bo, dummy, dummy, D, W, Pp, rows, False, False, co)

    o = o[:, :, 1:D + 1, :plane, :].reshape(B, 2, D, W2, W2, co)
    o = o[:, :, :, 1:W + 1, 1:W + 1, :]
    cls = jnp.transpose(o[:, 0, :, :, :, :nco], (0, 4, 1, 2, 3))
    reg = jnp.transpose(o[:, 1, :, :, :, :nro], (0, 4, 1, 2, 3))
    return cls, reg


def kernel(feat0, feat1, params):
    cls0, reg0 = _run_level(feat0, params, 20)
    cls1, reg1 = _run_level(feat1, params, 10)
    return (cls0, cls1, reg0, reg1)


# speedup vs baseline: 2.0223x; 1.2366x over previous
import jax
import jax.numpy as jnp
from jax.experimental import pallas as pl
from jax.experimental.pallas import tpu as pltpu

C = 128
G = 8
CG = 16
EPS = 1e-5
WP = 24


def _round8(n):
    return ((n + 7) // 8) * 8


def _make_layer_kernel(D, W, H2, Pp, rows, use_gn, cout):
    D2 = D + 2
    base = WP

    def kern(x_ref, w_ref, b_ref, g_ref, be_ref, o_ref, xp_ref, xm_ref):
        o_ref[...] = jnp.zeros_like(o_ref)

        def roll_body(p, _):
            plane = x_ref[0, 0, p]
            xp_ref[p] = jnp.roll(plane, 1, axis=0)
            xm_ref[p] = jnp.roll(plane, -1, axis=0)
            return 0

        jax.lax.fori_loop(0, D2, roll_body, 0)

        mask = ((jax.lax.broadcasted_iota(jnp.int32, (rows, 1), 0) % WP) < W)
        maskf = mask.astype(jnp.float32)
        bias = b_ref[0]

        def body(d, carry):
            s_c, q_c = carry
            acc = None
            for kd in range(3):
                p = d + kd
                wk = w_ref[0, kd]
                for kh in range(3):
                    off = kh * WP
                    for kw, src in ((0, xp_ref[p, pl.ds(off, rows), :]),
                                    (1, x_ref[0, 0, p, pl.ds(off, rows), :]),
                                    (2, xm_ref[p, pl.ds(off, rows), :])):
                        t = 3 * kh + kw
                        pp = jnp.dot(src, wk[t * C:(t + 1) * C, :],
                                     preferred_element_type=jnp.float32)
                        acc = pp if acc is None else acc + pp
            acc = (acc + bias) * maskf
            o_ref[0, 0, d + 1, pl.ds(base, rows), :] = acc
            s_c = s_c + jnp.sum(acc, axis=0, keepdims=True)
            q_c = q_c + jnp.sum(acc * acc, axis=0, keepdims=True)
            return s_c, q_c

        s_c, q_c = jax.lax.fori_loop(
            0, D, body,
            (jnp.zeros((1, cout), jnp.float32), jnp.zeros((1, cout), jnp.float32)))

        if use_gn:
            gi = jax.lax.broadcasted_iota(jnp.int32, (C, C), 0) // CG
            gj = jax.lax.broadcasted_iota(jnp.int32, (C, C), 1) // CG
            agg = (gi == gj).astype(jnp.float32)
            cnt = float(D * W * W * CG)
            gs = jnp.dot(s_c, agg, preferred_element_type=jnp.float32) / cnt
            gq = jnp.dot(q_c, agg, preferred_element_type=jnp.float32) / cnt
            var = gq - gs * gs
            inv = jax.lax.rsqrt(var + EPS)
            scale = g_ref[0]
            beta = be_ref[0]
            a = inv * scale
            bb = beta - gs * inv * scale

            def body2(d, _):
                y = o_ref[0, 0, d + 1, pl.ds(base, rows), :]
                y = jnp.maximum(y * a + bb, 0.0) * maskf
                o_ref[0, 0, d + 1, pl.ds(base, rows), :] = y
                return 0

            jax.lax.fori_loop(0, D, body2, 0)

    return kern


def _layer_call(x, w, b, g, be, D, W, H2, Pp, rows, use_gn, shared_input, cout):
    B = x.shape[0]
    T = w.shape[0]
    D2 = D + 2
    kern = _make_layer_kernel(D, W, H2, Pp, rows, use_gn, cout)
    if shared_input:
        x_spec = pl.BlockSpec((1, 1, D2, Pp, C), lambda bi, ti: (bi, 0, 0, 0, 0))
    else:
        x_spec = pl.BlockSpec((1, 1, D2, Pp, C), lambda bi, ti: (bi, ti, 0, 0, 0))
    w_spec = pl.BlockSpec((1, 3, 9 * C, cout), lambda bi, ti: (ti, 0, 0, 0))
    v_spec = pl.BlockSpec((1, 1, cout), lambda bi, ti: (ti, 0, 0))
    vC_spec = pl.BlockSpec((1, 1, C), lambda bi, ti: (ti, 0, 0))
    o_spec = pl.BlockSpec((1, 1, D2, Pp, cout), lambda bi, ti: (bi, ti, 0, 0, 0))
    return pl.pallas_call(
        kern,
        grid=(B, T),
        in_specs=[x_spec, w_spec, v_spec, vC_spec, vC_spec],
        out_specs=o_spec,
        out_shape=jax.ShapeDtypeStruct((B, T, D2, Pp, cout), jnp.float32),
        scratch_shapes=[pltpu.VMEM((D2, Pp, C), jnp.float32),
                        pltpu.VMEM((D2, Pp, C), jnp.float32)],
        compiler_params=pltpu.CompilerParams(
            dimension_semantics=("parallel", "parallel")),
    )(x, w, b, g, be)


def _prep_x(feat, D, W, Pp):
    B = feat.shape[0]
    x = jnp.transpose(feat, (0, 2, 3, 4, 1))
    x = jnp.pad(x, ((0, 0), (1, 1), (1, 1), (0, WP - W), (0, 0)))
    x = x.reshape(B, D + 2, (D + 2) * WP, C)
    return x[:, None]


def _prep_w(w):
    o, i = w.shape[0], w.shape[1]
    wt = jnp.transpose(w, (2, 3, 4, 1, 0))
    return wt.reshape(3, 9 * i, o)


def _run_level(feat, params, D):
    W = D
    H2 = D + 2
    Pp = H2 * WP
    rows = _round8((D - 1) * WP + W)
    B = feat.shape[0]

    x = _prep_x(feat, D, W, Pp)
    pc, pr = params['cls'], params['reg']
    for l in range(4):
        wc, bc, gc, bec = pc['conv'][l]
        wr, br, gr, ber = pr['conv'][l]
        w = jnp.stack([_prep_w(wc), _prep_w(wr)])
        b = jnp.stack([bc, br])[:, None, :]
        g = jnp.stack([gc, gr])[:, None, :]
        be = jnp.stack([bec, ber])[:, None, :]
        x = _layer_call(x, w, b, g, be, D, W, H2, Pp, rows, True, l == 0, C)

    (wco, boc), (wro, bor) = pc['out'], pr['out']
    nco, nro = wco.shape[0], wro.shape[0]
    co = 32
    wo = jnp.stack([
        jnp.pad(_prep_w(wco), ((0, 0), (0, 0), (0, co - nco))),
        jnp.pad(_prep_w(wro), ((0, 0), (0, 0), (0, co - nro))),
    ])
    bo = jnp.stack([jnp.pad(boc, (0, co - nco)), jnp.pad(bor, (0, co - nro))])
    bo = bo[:, None, :]
    dummy = jnp.zeros((2, 1, C), jnp.float32)
    o = _layer_call(x, wo, bo, dummy, dummy, D, W, H2, Pp, rows, False, False, co)

    o = o[:, :, 1:D + 1, :, :].reshape(B, 2, D, H2, WP, co)
    o = o[:, :, :, 1:W + 1, :W, :]
    cls = jnp.transpose(o[:, 0, :, :, :, :nco], (0, 4, 1, 2, 3))
    reg = jnp.transpose(o[:, 1, :, :, :, :nro], (0, 4, 1, 2, 3))
    return cls, reg


def kernel(feat0, feat1, params):
    cls0, reg0 = _run_level(feat0, params, 20)
    cls1, reg1 = _run_level(feat1, params, 10)
    return (cls0, cls1, reg0, reg1)
